# Initial kernel scaffold; baseline (speedup 1.0000x reference)
import jax
import jax.numpy as jnp
from jax.experimental import pallas as pl


def kernel(x, edge_index, W0_1, W1_1, b1, W0_2, W1_2, b2):
    raise NotImplementedError("write your pallas kernel here")


# speedup vs baseline: 12.5079x; 12.5079x over previous
import functools

import jax
import jax.numpy as jnp
from jax import lax
from jax.experimental import pallas as pl
from jax.experimental.pallas import tpu as pltpu
from jax.experimental.pallas import tpu_sc as plsc

N = 10000
E = 320000
F_IN = 128
F_HID = 128
F_OUT = 40
F2 = 64

NP = 10240
TRASH = NP - 1

NC = 2
NS = 16
NW = NC * NS
EPT = E // NW
CHUNK = 80
NCHUNK = EPT // CHUNK

ROWS_PT = NP // NS
RB = 80

_MESH = plsc.VectorSubcoreMesh(core_axis_name="c", subcore_axis_name="s")
_SC_PARAMS = pltpu.CompilerParams(needs_layout_passes=False)


HR = NP // 128
HRT = 8
HTS = HR // HRT


def _deg_body(src_hbm, dst_hbm, deg_out, mdst_out, srcv, dstv, mdstv, histv,
              zb, idx80, accdeg):
    c = lax.axis_index("c")
    s = lax.axis_index("s")
    wid = c * NS + s
    base = wid * EPT
    pltpu.sync_copy(src_hbm.at[pl.ds(base, EPT)], srcv)
    pltpu.sync_copy(dst_hbm.at[pl.ds(base, EPT)], dstv)

    io16 = lax.iota(jnp.int32, 16)
    zi = io16 * 0
    zf = zi.astype(jnp.float32)

    for r in range(HRT):
        for k in range(8):
            zb[r, pl.ds(k * 16, 16)] = zf
    for k in range(HR // 16):
        idx80[pl.ds(k * 16, 16)] = io16 + k * 16

    def zero_row(r, _):
        for k in range(8):
            histv[r, pl.ds(k * 16, 16)] = zf
        return 0

    lax.fori_loop(0, HR, zero_row, 0)

    @pl.when(s < HTS)
    def _():
        pltpu.sync_copy(zb, accdeg.at[pl.ds(s * HRT, HRT)])
    plsc.subcore_barrier()

    trash = zi + TRASH

    def edge_vec(i, _):
        sv = srcv[pl.ds(i * 16, 16)]
        dv = dstv[pl.ds(i * 16, 16)]
        m = sv != dv
        plsc.addupdate_scatter(
            histv,
            [lax.shift_right_logical(sv, 7), lax.bitwise_and(sv, 127)],
            m.astype(jnp.float32))
        mdstv[pl.ds(i * 16, 16)] = jnp.where(m, dv, trash)
        return 0

    lax.fori_loop(0, EPT // 16, edge_vec, 0)

    pltpu.sync_copy(histv, accdeg.at[idx80], add=True)
    plsc.subcore_barrier()

    @pl.when(s < HTS)
    def _():
        pltpu.sync_copy(accdeg.at[pl.ds(s * HRT, HRT)], zb)
        pltpu.sync_copy(zb, deg_out.at[c, pl.ds(s * HRT, HRT)])

    pltpu.sync_copy(mdstv, mdst_out.at[pl.ds(base, EPT)])


_deg_kernel = functools.partial(
    pl.kernel,
    out_type=[
        jax.ShapeDtypeStruct((NC, HR, 128), jnp.float32),
        jax.ShapeDtypeStruct((E,), jnp.int32),
    ],
    mesh=_MESH,
    scratch_types=[
        pltpu.VMEM((EPT,), jnp.int32),
        pltpu.VMEM((EPT,), jnp.int32),
        pltpu.VMEM((EPT,), jnp.int32),
        pltpu.VMEM((HR, 128), jnp.float32),
        pltpu.VMEM((HRT, 128), jnp.float32),
        pltpu.VMEM((HR,), jnp.int32),
        pltpu.VMEM_SHARED((HR, 128), jnp.float32),
    ],
    compiler_params=_SC_PARAMS,
)(_deg_body)


def _prop_body(width, y_hbm, src_hbm, mdst_hbm, p_out, srcv, mdstv, rows,
               zbuf, acc, sem):
    c = lax.axis_index("c")
    s = lax.axis_index("s")
    wid = c * NS + s

    zf = (lax.iota(jnp.int32, 16) * 0).astype(jnp.float32)

    def zero_row(r, _):
        for k in range(width // 16):
            zbuf[r, pl.ds(k * 16, 16)] = zf
        return 0

    lax.fori_loop(0, RB, zero_row, 0)
    for k in range(ROWS_PT // RB):
        pltpu.sync_copy(zbuf, acc.at[pl.ds(s * ROWS_PT + k * RB, RB)])
    plsc.subcore_barrier()

    def chunk(ci, _):
        base = wid * EPT + ci * CHUNK
        pltpu.sync_copy(src_hbm.at[pl.ds(base, CHUNK)], srcv)
        pltpu.sync_copy(mdst_hbm.at[pl.ds(base, CHUNK)], mdstv)
        pltpu.async_copy(y_hbm.at[srcv], rows, sem).wait()
        pltpu.sync_copy(rows, acc.at[mdstv], add=True)
        return 0

    lax.fori_loop(0, NCHUNK, chunk, 0)
    plsc.subcore_barrier()

    for k in range(ROWS_PT // RB):
        r0 = s * ROWS_PT + k * RB
        pltpu.sync_copy(acc.at[pl.ds(r0, RB)], zbuf)
        pltpu.sync_copy(zbuf, p_out.at[c, pl.ds(r0, RB)])


def _make_prop(width):
    params = (_SC_PARAMS if width % 128 == 0 else
              pltpu.CompilerParams(needs_layout_passes=False,
                                   use_tc_tiling_on_sc=False))
    return functools.partial(
        pl.kernel,
        out_type=jax.ShapeDtypeStruct((NC, NP, width), jnp.float32),
        mesh=_MESH,
        scratch_types=[
            pltpu.VMEM((CHUNK,), jnp.int32),
            pltpu.VMEM((CHUNK,), jnp.int32),
            pltpu.VMEM((CHUNK, width), jnp.float32),
            pltpu.VMEM((RB, width), jnp.float32),
            pltpu.VMEM_SHARED((NP, width), jnp.float32),
            pltpu.SemaphoreType.DMA,
        ],
        compiler_params=params,
    )(functools.partial(_prop_body, width))


_prop128 = _make_prop(F_HID)
_prop64 = _make_prop(F2)


def _s1_body(x_ref, w0_ref, w1_ref, dp_ref, xw0_ref, y1_ref, dis_ref):
    deg = jnp.sum(dp_ref[...], axis=0)
    dis = jnp.where(deg > 0, lax.rsqrt(jnp.maximum(deg, 1e-12)), 0.0)
    xb = x_ref[...]
    xw0_ref[...] = jnp.dot(xb, w0_ref[...], preferred_element_type=jnp.float32)
    y1_ref[...] = dis * jnp.dot(xb, w1_ref[...],
                                preferred_element_type=jnp.float32)
    dis_ref[...] = dis


def _s2_body(xw0_ref, p_ref, dis_ref, b1_ref, w02_ref, w12_ref,
             hw0_ref, y2_ref):
    dis = dis_ref[...]
    h = jnp.maximum(
        xw0_ref[...] - dis * (p_ref[0] + p_ref[1]) + b1_ref[...], 0.0)
    hw0_ref[...] = jnp.dot(h, w02_ref[...], preferred_element_type=jnp.float32)
    y2_ref[...] = dis * jnp.dot(h, w12_ref[...],
                                preferred_element_type=jnp.float32)


def _s3_body(hw0_ref, q_ref, dis_ref, b2_ref, out_ref):
    out_ref[...] = (hw0_ref[:, :F_OUT]
                    - dis_ref[...] * (q_ref[0, :, :F_OUT] + q_ref[1, :, :F_OUT])
                    + b2_ref[...])


BR = 2048
BR3 = 2000


def _stage1(x_p, W0, W1, dp):
    return pl.pallas_call(
        _s1_body,
        grid=(NP // BR,),
        in_specs=[
            pl.BlockSpec((BR, F_IN), lambda i: (i, 0)),
            pl.BlockSpec((F_IN, F_HID), lambda i: (0, 0)),
            pl.BlockSpec((F_IN, F_HID), lambda i: (0, 0)),
            pl.BlockSpec((NC, BR, 1), lambda i: (0, i, 0)),
        ],
        out_specs=[
            pl.BlockSpec((BR, F_HID), lambda i: (i, 0)),
            pl.BlockSpec((BR, F_HID), lambda i: (i, 0)),
            pl.BlockSpec((BR, 1), lambda i: (i, 0)),
        ],
        out_shape=[
            jax.ShapeDtypeStruct((NP, F_HID), jnp.float32),
            jax.ShapeDtypeStruct((NP, F_HID), jnp.float32),
            jax.ShapeDtypeStruct((NP, 1), jnp.float32),
        ],
    )(x_p, W0, W1, dp)


def _stage2(xw0, p, dis, b1r, W02p, W12p):
    return pl.pallas_call(
        _s2_body,
        grid=(NP // BR,),
        in_specs=[
            pl.BlockSpec((BR, F_HID), lambda i: (i, 0)),
            pl.BlockSpec((NC, BR, F_HID), lambda i: (0, i, 0)),
            pl.BlockSpec((BR, 1), lambda i: (i, 0)),
            pl.BlockSpec((1, F_HID), lambda i: (0, 0)),
            pl.BlockSpec((F_HID, F2), lambda i: (0, 0)),
            pl.BlockSpec((F_HID, F2), lambda i: (0, 0)),
        ],
        out_specs=[
            pl.BlockSpec((BR, F2), lambda i: (i, 0)),
            pl.BlockSpec((BR, F2), lambda i: (i, 0)),
        ],
        out_shape=[
            jax.ShapeDtypeStruct((NP, F2), jnp.float32),
            jax.ShapeDtypeStruct((NP, F2), jnp.float32),
        ],
    )(xw0, p, dis, b1r, W02p, W12p)


def _stage3(hw0, q, dis, b2r):
    return pl.pallas_call(
        _s3_body,
        grid=(N // BR3,),
        in_specs=[
            pl.BlockSpec((BR3, F2), lambda i: (i, 0)),
            pl.BlockSpec((NC, BR3, F2), lambda i: (0, i, 0)),
            pl.BlockSpec((BR3, 1), lambda i: (i, 0)),
            pl.BlockSpec((1, F_OUT), lambda i: (0, 0)),
        ],
        out_specs=pl.BlockSpec((BR3, F_OUT), lambda i: (i, 0)),
        out_shape=jax.ShapeDtypeStruct((N, F_OUT), jnp.float32),
    )(hw0, q, dis, b2r)


def kernel(x, edge_index, W0_1, W1_1, b1, W0_2, W1_2, b2):
    src = edge_index[0]
    dst = edge_index[1]
    x_p = jnp.pad(x, ((0, NP - N), (0, 0)))
    W02p = jnp.pad(W0_2, ((0, 0), (0, F2 - F_OUT)))
    W12p = jnp.pad(W1_2, ((0, 0), (0, F2 - F_OUT)))
    b1r = b1.reshape(1, F_HID)
    b2r = b2.reshape(1, F_OUT)

    deg_part, mdst = _deg_kernel(src, dst)
    xw0, y1, dis = _stage1(x_p, W0_1, W1_1, deg_part.reshape(NC, NP, 1))
    p = _prop128(y1, src, mdst)
    hw0, y2 = _stage2(xw0, p, dis, b1r, W02p, W12p)
    q = _prop64(y2, src, mdst)
    out = _stage3(hw0, q, dis, b2r)
    return (out, edge_index)
